# Initial kernel scaffold; baseline (speedup 1.0000x reference)
import jax
import jax.numpy as jnp
from jax.experimental import pallas as pl


def kernel(t, x, fragment_seq, atoms_rotamer, amino_acid_pos_rotamer, bond_matrix_rotamer, edge_mask_rotamer, atom_mask_rotamer, We1, be1, We2, be2, Wx, bx, Wh, bh):
    raise NotImplementedError("write your pallas kernel here")


# speedup vs baseline: 23.8472x; 23.8472x over previous
import jax
import jax.numpy as jnp
from jax import lax
from jax.experimental import pallas as pl
from jax.experimental.pallas import tpu as pltpu

_B, _S, _N, _L = 4, 4, 64, 15
_NUM_LAYERS = 2
_HDIM = 78
_HID = 128


def _fwd(t_ref, x_ref, frag_ref, atoms_ref, posc_ref, posr_ref, bond_ref,
         em_ref, am_ref, We1_ref, be1_ref, We2_ref, be2_ref, Wx_ref, bx_ref,
         Wh_ref, bh_ref, out_ref):
    f32 = jnp.float32
    N, L, HDIM = _N, _L, _HDIM

    x0 = x_ref[0, 0]
    am = am_ref[0]
    bond = bond_ref[0]
    em = em_ref[0]
    t = t_ref[0, 0, 0]
    atoms_col = atoms_ref[0]
    pos_col = posc_ref[0]
    pos_row = posr_ref[0]
    frag_col = frag_ref[0]

    eye = (lax.broadcasted_iota(jnp.int32, (N, N), 0) ==
           lax.broadcasted_iota(jnp.int32, (N, N), 1)).astype(f32)
    eye_h = (lax.broadcasted_iota(jnp.int32, (_HID, _HID), 0) ==
             lax.broadcasted_iota(jnp.int32, (_HID, _HID), 1)).astype(f32)

    pos_oh = (lax.broadcasted_iota(jnp.int32, (N, L), 1)
              == (pos_col - 1)).astype(f32)
    frag_oh = (lax.broadcasted_iota(jnp.int32, (L, 20), 1)
               == frag_col).astype(f32)
    aa_col = jnp.dot(pos_oh,
                     jnp.dot(frag_oh,
                             lax.broadcasted_iota(jnp.int32, (20, 1), 0).astype(f32),
                             preferred_element_type=f32),
                     preferred_element_type=f32)
    i78 = lax.broadcasted_iota(jnp.int32, (N, HDIM), 1)
    i78f = i78.astype(f32)
    h = ((i78 == atoms_col).astype(f32)
         + (i78f == aa_col + 43.0).astype(f32)
         + (i78 == (pos_col - 1) + 63).astype(f32))

    xc = [x0[:, c:c + 1] for c in range(3)]
    em_sum = jnp.sum(em, axis=1, keepdims=True)
    inv_n = 1.0 / N

    for i in range(_NUM_LAYERS):
        W1 = We1_ref[i]
        W1s = W1[:HDIM, :]
        W1d = W1[HDIM:2 * HDIM, :]
        wd = W1[2 * HDIM:2 * HDIM + 1, :]
        wb = W1[2 * HDIM + 1:2 * HDIM + 2, :]
        wt = W1[2 * HDIM + 2:2 * HDIM + 3, :]
        b1 = be1_ref[i:i + 1, :]
        W2 = We2_ref[i]
        b2 = be2_ref[i:i + 1, :]
        Wx_i = Wx_ref[i]
        bx_i = bx_ref[i, 0]
        Whh = Wh_ref[i]
        bh_i = bh_ref[i:i + 1, :]

        xr = [jnp.sum(eye * xc[c], axis=0, keepdims=True) for c in range(3)]
        d = [xc[c] - xr[c] for c in range(3)]
        dist = jnp.sqrt(d[0] * d[0] + d[1] * d[1] + d[2] * d[2] + 1e-12)

        A = jnp.dot(h, W1s, preferred_element_type=f32) + b1
        Bm = jnp.dot(h, W1d, preferred_element_type=f32)
        pre = (A[:, None, :] + Bm[None, :, :]
               + dist[:, :, None] * wd[None, :, :]
               + bond[:, :, None] * wb[None, :, :]
               + (t * wt)[None, :, :])
        R = jnp.maximum(pre, 0.0)

        S1 = jnp.sum(R * em[:, :, None], axis=1)
        h_agg = jnp.dot(S1, W2, preferred_element_type=f32) + em_sum * b2

        v = jnp.dot(W2, Wx_i, preferred_element_type=f32)
        c2 = jnp.dot(b2, Wx_i, preferred_element_type=f32)
        v_row = jnp.sum(eye_h * v, axis=0, keepdims=True)
        u = jnp.sum(R * v_row[None, :, :], axis=2)
        w = jnp.tanh(em * (u + c2[0, 0]) + bx_i)

        for c in range(3):
            xout = (jnp.sum(d[c] * w, axis=1, keepdims=True)
                    * inv_n * am[:, c:c + 1])
            xc[c] = xc[c] + xout

        h = jnp.tanh(jnp.dot(h, Whh[:HDIM, :], preferred_element_type=f32)
                     + jnp.dot(h_agg, Whh[HDIM:, :], preferred_element_type=f32)
                     + bh_i)

    seg_oh = pos_oh
    seg_ohT = (lax.broadcasted_iota(jnp.int32, (L, N), 0)
               == (pos_row - 1)).astype(f32)
    for c in range(3):
        am_c = am[:, c:c + 1]
        p_c = (xc[c] - x0[:, c:c + 1]) * am_c
        cm_c = jnp.dot(seg_ohT, p_c, preferred_element_type=f32)
        cnt_c = jnp.dot(seg_ohT, am_c, preferred_element_type=f32)
        mean_c = cm_c / (cnt_c + 1e-8)
        gath_c = jnp.dot(seg_oh, mean_c, preferred_element_type=f32)
        out_ref[0, 0, :, c:c + 1] = (p_c - gath_c) * am_c


def kernel(t, x, fragment_seq, atoms_rotamer, amino_acid_pos_rotamer,
           bond_matrix_rotamer, edge_mask_rotamer, atom_mask_rotamer,
           We1, be1, We2, be2, Wx, bx, Wh, bh):
    f32 = jnp.float32
    i32 = jnp.int32
    B, S, N, L = _B, _S, _N, _L

    t3 = t.astype(f32).reshape(B, 1, 1)
    frag_c = fragment_seq.astype(i32).reshape(B, L, 1)
    atoms_c = atoms_rotamer.astype(i32).reshape(B, N, 1)
    pos_c = amino_acid_pos_rotamer.astype(i32).reshape(B, N, 1)
    pos_r = amino_acid_pos_rotamer.astype(i32).reshape(B, 1, N)

    const = lambda *shape: (lambda b, s: tuple(0 for _ in shape))
    per_b = lambda ndim: (lambda b, s: (b,) + (0,) * (ndim - 1))

    in_specs = [
        pl.BlockSpec((1, 1, 1), per_b(3)),
        pl.BlockSpec((1, 1, N, 3), lambda b, s: (b, s, 0, 0)),
        pl.BlockSpec((1, L, 1), per_b(3)),
        pl.BlockSpec((1, N, 1), per_b(3)),
        pl.BlockSpec((1, N, 1), per_b(3)),
        pl.BlockSpec((1, 1, N), per_b(3)),
        pl.BlockSpec((1, N, N), per_b(3)),
        pl.BlockSpec((1, N, N), per_b(3)),
        pl.BlockSpec((1, N, 3), per_b(3)),
        pl.BlockSpec(We1.shape, const(*We1.shape)),
        pl.BlockSpec(be1.shape, const(*be1.shape)),
        pl.BlockSpec(We2.shape, const(*We2.shape)),
        pl.BlockSpec(be2.shape, const(*be2.shape)),
        pl.BlockSpec(Wx.shape, const(*Wx.shape)),
        pl.BlockSpec(bx.shape, const(*bx.shape)),
        pl.BlockSpec(Wh.shape, const(*Wh.shape)),
        pl.BlockSpec(bh.shape, const(*bh.shape)),
    ]

    return pl.pallas_call(
        _fwd,
        grid=(B, S),
        in_specs=in_specs,
        out_specs=pl.BlockSpec((1, 1, N, 3), lambda b, s: (b, s, 0, 0)),
        out_shape=jax.ShapeDtypeStruct((B, S, N, 3), f32),
        compiler_params=pltpu.CompilerParams(
            dimension_semantics=("parallel", "parallel")),
    )(t3, x.astype(f32), frag_c, atoms_c, pos_c, pos_r,
      bond_matrix_rotamer.astype(f32), edge_mask_rotamer.astype(f32),
      atom_mask_rotamer.astype(f32),
      We1, be1, We2, be2, Wx, bx, Wh, bh)
